# Initial kernel scaffold; baseline (speedup 1.0000x reference)
import jax
import jax.numpy as jnp
from jax.experimental import pallas as pl


def kernel(x, edge_index, batch, params):
    raise NotImplementedError("write your pallas kernel here")


# speedup vs baseline: 3.0103x; 3.0103x over previous
import functools

import jax
import jax.numpy as jnp
from jax import lax
from jax.experimental import pallas as pl
from jax.experimental.pallas import tpu as pltpu
from jax.experimental.pallas import tpu_sc as plsc

NN = 50000
NPAD = 50176
EE = 800000
EB = 128
NBLK = 6400
EPAD = NBLK * EB
ROWS = 1024
GRID = NPAD // ROWS
NC, NS = 2, 16
BPT = NBLK // NS
RPT = NPAD // NS
ZR = 392
NG = 64


def _sc_agg(C):
    cpc = C // NC

    mesh = plsc.VectorSubcoreMesh(
        core_axis_name="c", subcore_axis_name="s", num_cores=NC, num_subcores=NS
    )

    @functools.partial(
        pl.kernel,
        out_type=[jax.ShapeDtypeStruct((NPAD, 32), jnp.float32)] * C,
        mesh=mesh,
        compiler_params=pltpu.CompilerParams(use_tc_tiling_on_sc=False),
        scratch_types=[
            pltpu.VMEM((16, EB), jnp.int32),
            pltpu.VMEM((16, EB), jnp.int32),
            pltpu.VMEM((EB, 32), jnp.float32),
            pltpu.VMEM((EB, 32), jnp.float32),
            pltpu.VMEM((ZR, 32), jnp.float32),
            pltpu.VMEM_SHARED((NPAD, 32), jnp.float32),
            pltpu.SemaphoreType.DMA,
            pltpu.SemaphoreType.DMA,
        ],
    )
    def agg_kernel(src_hbm, dst_hbm, *rest):
        h_refs = rest[:C]
        out_refs = rest[C:2 * C]
        sbuf, dbuf, rows_a, rows_b, zbuf, acc, sem_a, sem_b = rest[2 * C:]

        cid = lax.axis_index("c")
        sid = lax.axis_index("s")

        zero16 = jnp.zeros((16,), jnp.float32)

        def zinit(i, carry):
            zbuf[i, pl.ds(0, 16)] = zero16
            zbuf[i, pl.ds(16, 16)] = zero16
            return carry

        lax.fori_loop(0, ZR, zinit, 0)

        for c in range(C):

            @pl.when(cid == c // cpc)
            def _(c=c):
                h_ref = h_refs[c]
                for z in range(RPT // ZR):
                    pltpu.sync_copy(
                        zbuf, acc.at[pl.ds(sid * RPT + z * ZR, ZR)]
                    )
                plsc.subcore_barrier()

                def outer(o, carry):
                    base = sid * BPT + o * 16
                    pltpu.sync_copy(src_hbm.at[pl.ds(base, 16)], sbuf)
                    pltpu.sync_copy(dst_hbm.at[pl.ds(base, 16)], dbuf)

                    def inner(p, carry2):
                        j0 = 2 * p
                        j1 = 2 * p + 1
                        cp_a = pltpu.async_copy(
                            h_ref.at[sbuf.at[j0]], rows_a, sem_a
                        )
                        cp_b = pltpu.async_copy(
                            h_ref.at[sbuf.at[j1]], rows_b, sem_b
                        )
                        cp_a.wait()
                        pltpu.sync_copy(rows_a, acc.at[dbuf.at[j0]], add=True)
                        cp_b.wait()
                        pltpu.sync_copy(rows_b, acc.at[dbuf.at[j1]], add=True)
                        return carry2

                    lax.fori_loop(0, 8, inner, 0)
                    return carry

                lax.fori_loop(0, BPT // 16, outer, 0)
                plsc.subcore_barrier()

                pltpu.sync_copy(
                    acc.at[pl.ds(sid * RPT, RPT)],
                    out_refs[c].at[pl.ds(sid * RPT, RPT)],
                )
                plsc.subcore_barrier()

        return None

    return agg_kernel


def _mlp_stats(C, h_list, agg_list, wa, ba, wb, bb):
    def body(*refs):
        h_refs = refs[:C]
        a_refs = refs[C:2 * C]
        wa_ref, ba_ref, wb_ref, bb_ref, t_ref, st_ref = refs[2 * C:]
        i = pl.program_id(0)

        acc = jnp.zeros((ROWS, 128), jnp.float32)
        for c in range(C):
            m_c = h_refs[c][...] + a_refs[c][...]
            acc = acc + jnp.dot(
                m_c,
                wa_ref[pl.ds(c * 32, 32), :],
                preferred_element_type=jnp.float32,
            )
        z = jnp.maximum(acc + ba_ref[0, :], 0.0)
        t = (
            jnp.dot(z, wb_ref[...], preferred_element_type=jnp.float32)
            + bb_ref[0, :]
        )
        t_ref[...] = t

        rowid = i * ROWS + lax.broadcasted_iota(jnp.int32, (ROWS, 1), 0)
        tm = jnp.where(rowid < NN, t, 0.0)
        s = jnp.sum(tm, axis=0)
        ss = jnp.sum(tm * tm, axis=0)
        st = jnp.concatenate([s[None, :], ss[None, :]], axis=0)

        @pl.when(i == 0)
        def _():
            st_ref[...] = jnp.zeros((2, 128), jnp.float32)

        st_ref[...] += st

    chunk_spec = pl.BlockSpec((ROWS, 32), lambda i: (i, 0))
    out = pl.pallas_call(
        body,
        grid=(GRID,),
        in_specs=(
            [chunk_spec] * (2 * C)
            + [
                pl.BlockSpec((C * 32, 128), lambda i: (0, 0)),
                pl.BlockSpec((1, 128), lambda i: (0, 0)),
                pl.BlockSpec((128, 128), lambda i: (0, 0)),
                pl.BlockSpec((1, 128), lambda i: (0, 0)),
            ]
        ),
        out_specs=[
            pl.BlockSpec((ROWS, 128), lambda i: (i, 0)),
            pl.BlockSpec((2, 128), lambda i: (0, 0)),
        ],
        out_shape=[
            jax.ShapeDtypeStruct((NPAD, 128), jnp.float32),
            jax.ShapeDtypeStruct((2, 128), jnp.float32),
        ],
    )(*h_list, *agg_list, wa, ba, wb, bb)
    return out


def _bn_relu_chunk(t, scale, shift):
    def body(t_ref, sc_ref, sh_ref, *out_refs):
        h = jnp.maximum(t_ref[...] * sc_ref[0, :] + sh_ref[0, :], 0.0)
        for c in range(4):
            out_refs[c][...] = h[:, c * 32:(c + 1) * 32]

    chunk_spec = pl.BlockSpec((ROWS, 32), lambda i: (i, 0))
    return pl.pallas_call(
        body,
        grid=(GRID,),
        in_specs=[
            pl.BlockSpec((ROWS, 128), lambda i: (i, 0)),
            pl.BlockSpec((1, 128), lambda i: (0, 0)),
            pl.BlockSpec((1, 128), lambda i: (0, 0)),
        ],
        out_specs=[chunk_spec] * 4,
        out_shape=[jax.ShapeDtypeStruct((NPAD, 32), jnp.float32)] * 4,
    )(t, scale, shift)


def _bn_relu_pool(t, scale, shift, batch3d):
    def body(t_ref, sc_ref, sh_ref, b_ref, pool_ref):
        i = pl.program_id(0)
        h = jnp.maximum(t_ref[...] * sc_ref[0, :] + sh_ref[0, :], 0.0)
        seg = b_ref[0, 0, :]
        onehot = (
            lax.broadcasted_iota(jnp.int32, (NG, ROWS), 0) == seg[None, :]
        ).astype(jnp.float32)
        part = jnp.dot(onehot, h, preferred_element_type=jnp.float32)

        @pl.when(i == 0)
        def _():
            pool_ref[...] = jnp.zeros((NG, 128), jnp.float32)

        pool_ref[...] += part

    return pl.pallas_call(
        body,
        grid=(GRID,),
        in_specs=[
            pl.BlockSpec((ROWS, 128), lambda i: (i, 0)),
            pl.BlockSpec((1, 128), lambda i: (0, 0)),
            pl.BlockSpec((1, 128), lambda i: (0, 0)),
            pl.BlockSpec((1, 1, ROWS), lambda i: (i, 0, 0)),
        ],
        out_specs=pl.BlockSpec((NG, 128), lambda i: (0, 0)),
        out_shape=jax.ShapeDtypeStruct((NG, 128), jnp.float32),
    )(t, scale, shift, batch3d)


def _head(pooled, wc1, bc1, wc2, bc2):
    def body(p_ref, w1_ref, b1_ref, w2_ref, b2_ref, o_ref):
        z1 = jnp.maximum(
            jnp.dot(p_ref[...], w1_ref[...], preferred_element_type=jnp.float32)
            + b1_ref[0, :],
            0.0,
        )
        z = (
            jnp.dot(z1, w2_ref[...], preferred_element_type=jnp.float32)
            + b2_ref[0, :]
        )
        zmax = jnp.max(z, axis=1, keepdims=True)
        lse = zmax + jnp.log(jnp.sum(jnp.exp(z - zmax), axis=1, keepdims=True))
        o_ref[...] = z - lse

    return pl.pallas_call(
        body,
        out_shape=jax.ShapeDtypeStruct((NG, 2), jnp.float32),
    )(pooled, wc1, bc1, wc2, bc2)


def kernel(x, edge_index, batch, params):
    layers, head = params
    src = edge_index[0].astype(jnp.int32)
    dst = edge_index[1].astype(jnp.int32)

    src2d = jnp.concatenate(
        [src, jnp.zeros((EPAD - EE,), jnp.int32)]
    ).reshape(NBLK, EB)
    dst2d = jnp.concatenate(
        [dst, jnp.full((EPAD - EE,), NN, jnp.int32)]
    ).reshape(NBLK, EB)

    batch3d = jnp.concatenate(
        [batch.astype(jnp.int32), jnp.full((NPAD - NN,), NG, jnp.int32)]
    ).reshape(GRID, 1, ROWS)

    h_list = [
        jnp.pad(x[:, 32 * c:32 * (c + 1)], ((0, NPAD - NN), (0, 0)))
        for c in range(2)
    ]

    pooled = None
    for li, (wa, ba, wb, bb, gamma, beta) in enumerate(layers):
        C = len(h_list)
        agg_list = _sc_agg(C)(src2d, dst2d, *h_list)
        t, st = _mlp_stats(
            C,
            h_list,
            agg_list,
            wa,
            ba.reshape(1, 128),
            wb,
            bb.reshape(1, 128),
        )
        mean = st[0] / NN
        var = st[1] / NN - mean * mean
        inv = gamma * lax.rsqrt(var + 1e-5)
        scale = inv.reshape(1, 128)
        shift = (beta - mean * inv).reshape(1, 128)
        if li < 2:
            h_list = _bn_relu_chunk(t, scale, shift)
        else:
            pooled = _bn_relu_pool(t, scale, shift, batch3d)

    wc1, bc1, wc2, bc2 = head
    return _head(
        pooled, wc1, bc1.reshape(1, 64), wc2, bc2.reshape(1, 2)
    )


# speedup vs baseline: 3.6674x; 1.2183x over previous
import functools

import jax
import jax.numpy as jnp
from jax import lax
from jax.experimental import pallas as pl
from jax.experimental.pallas import tpu as pltpu
from jax.experimental.pallas import tpu_sc as plsc

NN = 50000
NPAD = 50176
EE = 800000
EB = 128
NBLK = 6400
EPAD = NBLK * EB
ROWS = 1024
GRID = NPAD // ROWS
NC, NS = 2, 16
BPT = NBLK // NS
RPT = NPAD // NS
ZR = 98
NG = 64
CHK = 16
NCHK = BPT // CHK
DEPTH = 4


def _sc_agg(C):
    cpc = C // NC

    mesh = plsc.VectorSubcoreMesh(
        core_axis_name="c", subcore_axis_name="s", num_cores=NC, num_subcores=NS
    )

    @functools.partial(
        pl.kernel,
        out_type=[jax.ShapeDtypeStruct((NPAD, 32), jnp.float32)] * C,
        mesh=mesh,
        compiler_params=pltpu.CompilerParams(use_tc_tiling_on_sc=False),
        scratch_types=(
            [
                pltpu.VMEM((2, CHK, EB), jnp.int32),
                pltpu.VMEM((2, CHK, EB), jnp.int32),
                pltpu.VMEM((ZR, 32), jnp.float32),
                pltpu.VMEM_SHARED((NPAD, 32), jnp.float32),
            ]
            + [pltpu.VMEM((EB, 32), jnp.float32) for _ in range(DEPTH)]
            + [pltpu.SemaphoreType.DMA] * (2 + 2 * DEPTH)
        ),
    )
    def agg_kernel(src_hbm, dst_hbm, *rest):
        h_refs = rest[:C]
        out_refs = rest[C:2 * C]
        rest = rest[2 * C:]
        sbuf, dbuf, zbuf, acc = rest[:4]
        rows = rest[4:4 + DEPTH]
        sem_si, sem_di = rest[4 + DEPTH:6 + DEPTH]
        sem_g = rest[6 + DEPTH:6 + 2 * DEPTH]
        sem_s = rest[6 + 2 * DEPTH:6 + 3 * DEPTH]

        cid = lax.axis_index("c")
        sid = lax.axis_index("s")

        zero16 = jnp.zeros((16,), jnp.float32)

        def zinit(i, carry):
            zbuf[i, pl.ds(0, 16)] = zero16
            zbuf[i, pl.ds(16, 16)] = zero16
            return carry

        lax.fori_loop(0, ZR, zinit, 0)

        for c in range(C):

            @pl.when(cid == c // cpc)
            def _(c=c):
                h_ref = h_refs[c]
                for z in range(RPT // ZR):
                    pltpu.sync_copy(
                        zbuf, acc.at[pl.ds(sid * RPT + z * ZR, ZR)]
                    )
                plsc.subcore_barrier()

                def fire_gather(b, par, j):
                    return pltpu.async_copy(
                        h_ref.at[sbuf.at[par, j]], rows[b], sem_g[b]
                    )

                def wait_gather(b, par, j):
                    pltpu.make_async_copy(
                        h_ref.at[sbuf.at[par, j]], rows[b], sem_g[b]
                    ).wait()

                def fire_scatter(b, par, j):
                    return pltpu.async_copy(
                        rows[b], acc.at[dbuf.at[par, j]], sem_s[b], add=True
                    )

                def wait_scatter(b, par, j):
                    pltpu.make_async_copy(
                        rows[b], acc.at[dbuf.at[par, j]], sem_s[b]
                    ).wait()

                def fire_idx(k, slot):
                    base = sid * BPT + k * CHK
                    pltpu.async_copy(
                        src_hbm.at[pl.ds(base, CHK)], sbuf.at[slot], sem_si
                    )
                    pltpu.async_copy(
                        dst_hbm.at[pl.ds(base, CHK)], dbuf.at[slot], sem_di
                    )

                def wait_idx(slot):
                    pltpu.make_async_copy(
                        src_hbm.at[pl.ds(0, CHK)], sbuf.at[slot], sem_si
                    ).wait()
                    pltpu.make_async_copy(
                        dst_hbm.at[pl.ds(0, CHK)], dbuf.at[slot], sem_di
                    ).wait()

                base0 = sid * BPT
                pltpu.sync_copy(src_hbm.at[pl.ds(base0, CHK)], sbuf.at[0])
                pltpu.sync_copy(dst_hbm.at[pl.ds(base0, CHK)], dbuf.at[0])
                fire_idx(1, 1)

                def chunk_body(k, carry2):
                    par = k % 2

                    @pl.when(k > 0)
                    def _():
                        wait_idx(par)

                    @pl.when(jnp.logical_and(k > 0, k < NCHK - 1))
                    def _():
                        fire_idx(k + 1, (k + 1) % 2)

                    for b in range(DEPTH):
                        fire_gather(b, par, b)
                    for g in range(CHK // DEPTH):
                        for b in range(DEPTH):
                            wait_gather(b, par, g * DEPTH + b)
                            fire_scatter(b, par, g * DEPTH + b)
                        for b in range(DEPTH):
                            wait_scatter(b, par, g * DEPTH + b)
                            if g < CHK // DEPTH - 1:
                                fire_gather(b, par, (g + 1) * DEPTH + b)
                    return carry2

                lax.fori_loop(0, NCHK, chunk_body, 0)
                plsc.subcore_barrier()

                pltpu.sync_copy(
                    acc.at[pl.ds(sid * RPT, RPT)],
                    out_refs[c].at[pl.ds(sid * RPT, RPT)],
                )
                plsc.subcore_barrier()

        return None

    return agg_kernel


def _mlp_stats(C, h_list, agg_list, wa, ba, wb, bb):
    def body(*refs):
        h_refs = refs[:C]
        a_refs = refs[C:2 * C]
        wa_ref, ba_ref, wb_ref, bb_ref, t_ref, st_ref = refs[2 * C:]
        i = pl.program_id(0)

        acc = jnp.zeros((ROWS, 128), jnp.float32)
        for c in range(C):
            m_c = h_refs[c][...] + a_refs[c][...]
            acc = acc + jnp.dot(
                m_c,
                wa_ref[pl.ds(c * 32, 32), :],
                preferred_element_type=jnp.float32,
            )
        z = jnp.maximum(acc + ba_ref[0, :], 0.0)
        t = (
            jnp.dot(z, wb_ref[...], preferred_element_type=jnp.float32)
            + bb_ref[0, :]
        )
        t_ref[...] = t

        rowid = i * ROWS + lax.broadcasted_iota(jnp.int32, (ROWS, 1), 0)
        tm = jnp.where(rowid < NN, t, 0.0)
        s = jnp.sum(tm, axis=0)
        ss = jnp.sum(tm * tm, axis=0)
        st = jnp.concatenate([s[None, :], ss[None, :]], axis=0)

        @pl.when(i == 0)
        def _():
            st_ref[...] = jnp.zeros((2, 128), jnp.float32)

        st_ref[...] += st

    chunk_spec = pl.BlockSpec((ROWS, 32), lambda i: (i, 0))
    out = pl.pallas_call(
        body,
        grid=(GRID,),
        in_specs=(
            [chunk_spec] * (2 * C)
            + [
                pl.BlockSpec((C * 32, 128), lambda i: (0, 0)),
                pl.BlockSpec((1, 128), lambda i: (0, 0)),
                pl.BlockSpec((128, 128), lambda i: (0, 0)),
                pl.BlockSpec((1, 128), lambda i: (0, 0)),
            ]
        ),
        out_specs=[
            pl.BlockSpec((ROWS, 128), lambda i: (i, 0)),
            pl.BlockSpec((2, 128), lambda i: (0, 0)),
        ],
        out_shape=[
            jax.ShapeDtypeStruct((NPAD, 128), jnp.float32),
            jax.ShapeDtypeStruct((2, 128), jnp.float32),
        ],
    )(*h_list, *agg_list, wa, ba, wb, bb)
    return out


def _bn_relu_chunk(t, scale, shift):
    def body(t_ref, sc_ref, sh_ref, *out_refs):
        h = jnp.maximum(t_ref[...] * sc_ref[0, :] + sh_ref[0, :], 0.0)
        for c in range(4):
            out_refs[c][...] = h[:, c * 32:(c + 1) * 32]

    chunk_spec = pl.BlockSpec((ROWS, 32), lambda i: (i, 0))
    return pl.pallas_call(
        body,
        grid=(GRID,),
        in_specs=[
            pl.BlockSpec((ROWS, 128), lambda i: (i, 0)),
            pl.BlockSpec((1, 128), lambda i: (0, 0)),
            pl.BlockSpec((1, 128), lambda i: (0, 0)),
        ],
        out_specs=[chunk_spec] * 4,
        out_shape=[jax.ShapeDtypeStruct((NPAD, 32), jnp.float32)] * 4,
    )(t, scale, shift)


def _bn_relu_pool(t, scale, shift, batch3d):
    def body(t_ref, sc_ref, sh_ref, b_ref, pool_ref):
        i = pl.program_id(0)
        h = jnp.maximum(t_ref[...] * sc_ref[0, :] + sh_ref[0, :], 0.0)
        seg = b_ref[0, 0, :]
        onehot = (
            lax.broadcasted_iota(jnp.int32, (NG, ROWS), 0) == seg[None, :]
        ).astype(jnp.float32)
        part = jnp.dot(onehot, h, preferred_element_type=jnp.float32)

        @pl.when(i == 0)
        def _():
            pool_ref[...] = jnp.zeros((NG, 128), jnp.float32)

        pool_ref[...] += part

    return pl.pallas_call(
        body,
        grid=(GRID,),
        in_specs=[
            pl.BlockSpec((ROWS, 128), lambda i: (i, 0)),
            pl.BlockSpec((1, 128), lambda i: (0, 0)),
            pl.BlockSpec((1, 128), lambda i: (0, 0)),
            pl.BlockSpec((1, 1, ROWS), lambda i: (i, 0, 0)),
        ],
        out_specs=pl.BlockSpec((NG, 128), lambda i: (0, 0)),
        out_shape=jax.ShapeDtypeStruct((NG, 128), jnp.float32),
    )(t, scale, shift, batch3d)


def _head(pooled, wc1, bc1, wc2, bc2):
    def body(p_ref, w1_ref, b1_ref, w2_ref, b2_ref, o_ref):
        z1 = jnp.maximum(
            jnp.dot(p_ref[...], w1_ref[...], preferred_element_type=jnp.float32)
            + b1_ref[0, :],
            0.0,
        )
        z = (
            jnp.dot(z1, w2_ref[...], preferred_element_type=jnp.float32)
            + b2_ref[0, :]
        )
        zmax = jnp.max(z, axis=1, keepdims=True)
        lse = zmax + jnp.log(jnp.sum(jnp.exp(z - zmax), axis=1, keepdims=True))
        o_ref[...] = z - lse

    return pl.pallas_call(
        body,
        out_shape=jax.ShapeDtypeStruct((NG, 2), jnp.float32),
    )(pooled, wc1, bc1, wc2, bc2)


def kernel(x, edge_index, batch, params):
    layers, head = params
    src = edge_index[0].astype(jnp.int32)
    dst = edge_index[1].astype(jnp.int32)

    src2d = jnp.concatenate(
        [src, jnp.zeros((EPAD - EE,), jnp.int32)]
    ).reshape(NBLK, EB)
    dst2d = jnp.concatenate(
        [dst, jnp.full((EPAD - EE,), NN, jnp.int32)]
    ).reshape(NBLK, EB)

    batch3d = jnp.concatenate(
        [batch.astype(jnp.int32), jnp.full((NPAD - NN,), NG, jnp.int32)]
    ).reshape(GRID, 1, ROWS)

    h_list = [
        jnp.pad(x[:, 32 * c:32 * (c + 1)], ((0, NPAD - NN), (0, 0)))
        for c in range(2)
    ]

    pooled = None
    for li, (wa, ba, wb, bb, gamma, beta) in enumerate(layers):
        C = len(h_list)
        agg_list = _sc_agg(C)(src2d, dst2d, *h_list)
        t, st = _mlp_stats(
            C,
            h_list,
            agg_list,
            wa,
            ba.reshape(1, 128),
            wb,
            bb.reshape(1, 128),
        )
        mean = st[0] / NN
        var = st[1] / NN - mean * mean
        inv = gamma * lax.rsqrt(var + 1e-5)
        scale = inv.reshape(1, 128)
        shift = (beta - mean * inv).reshape(1, 128)
        if li < 2:
            h_list = _bn_relu_chunk(t, scale, shift)
        else:
            pooled = _bn_relu_pool(t, scale, shift, batch3d)

    wc1, bc1, wc2, bc2 = head
    return _head(
        pooled, wc1, bc1.reshape(1, 64), wc2, bc2.reshape(1, 2)
    )
